# baseline (speedup 1.0000x reference)
import functools

import jax
import jax.numpy as jnp
from jax import lax
from jax.experimental import pallas as pl
from jax.experimental.pallas import tpu as pltpu
from jax.experimental.pallas import tpu_sc as plsc

S = 2048
D = 1024
E = 16
EH = E // 2
F = 1024
C = 128
EC = E * C
HC = EC // 2
RB = 512
NR = S // RB
NC = 2
NS = 16
NW = NC * NS
SPW = HC // NW

_GMASK = -4096


def _router_body(x_ref, wg_ref, pk_ref, laux_ref, cnt_ref, qc_ref,
                 carry_ref, acc_ref, dcarry_ref):
    i = pl.program_id(0)

    @pl.when(i == 0)
    def _():
        carry_ref[...] = jnp.zeros((1, E), jnp.float32)
        acc_ref[...] = jnp.zeros((1, E), jnp.float32)
        dcarry_ref[...] = jnp.zeros((1, 1), jnp.float32)

    x = x_ref[...]
    wg = wg_ref[...]
    logits = jnp.dot(x, wg, preferred_element_type=jnp.float32)
    mx = jnp.max(logits, axis=1, keepdims=True)
    p = jnp.exp(logits - mx)
    gates = p / jnp.sum(p, axis=1, keepdims=True)
    gmax = jnp.max(gates, axis=1, keepdims=True)
    ie = lax.broadcasted_iota(jnp.int32, (RB, E), 1)
    idx1 = jnp.min(jnp.where(gates == gmax, ie, E), axis=1, keepdims=True)
    oh = (ie == idx1).astype(jnp.float32)

    carry = carry_ref[...]
    tri = (lax.broadcasted_iota(jnp.int32, (RB, RB), 0) >=
           lax.broadcasted_iota(jnp.int32, (RB, RB), 1)).astype(jnp.float32)
    incl = jnp.dot(tri, oh, preferred_element_type=jnp.float32) + carry
    pos = incl - 1.0
    pos_s = jnp.sum(pos * oh, axis=1, keepdims=True)
    kept = pos_s < C

    dmask = jnp.where(kept, 0.0, 1.0)
    dincl = (jnp.dot(tri, dmask, preferred_element_type=jnp.float32)
             + dcarry_ref[...])
    drank = (dincl - 1.0).astype(jnp.int32)
    dcarry_ref[...] = dcarry_ref[...] + jnp.sum(dmask, axis=0, keepdims=True)

    gbits = lax.bitcast_convert_type(gmax, jnp.int32) & _GMASK
    slot = idx1 * C + pos_s.astype(jnp.int32)
    pk_ref[...] = jnp.where(kept, gbits | slot, (drank << 12) | 4095)

    counts = carry + jnp.sum(oh, axis=0, keepdims=True)
    carry_ref[...] = counts
    me = acc_ref[...] + jnp.sum(gates, axis=0, keepdims=True)
    acc_ref[...] = me

    @pl.when(i == NR - 1)
    def _():
        cnt_post = jnp.minimum(counts, C)
        cnt_ref[...] = cnt_post.astype(jnp.int32)
        laux_ref[...] = jnp.sum(me * counts, axis=1,
                                keepdims=True) * (E / (S * S))
        ut = (lax.broadcasted_iota(jnp.int32, (E, E), 0) <=
              lax.broadcasted_iota(jnp.int32, (E, E), 1)).astype(jnp.float32)
        q = jnp.dot(cnt_post, ut,
                    preferred_element_type=jnp.float32).astype(jnp.int32)
        qc_ref[...] = (q << 8) | cnt_post.astype(jnp.int32)


_router = pl.pallas_call(
    _router_body,
    grid=(NR,),
    in_specs=[
        pl.BlockSpec((RB, D), lambda i: (i, 0)),
        pl.BlockSpec((D, E), lambda i: (0, 0)),
    ],
    out_specs=[
        pl.BlockSpec((RB, 1), lambda i: (i, 0)),
        pl.BlockSpec((1, 1), lambda i: (0, 0)),
        pl.BlockSpec((1, E), lambda i: (0, 0)),
        pl.BlockSpec((1, E), lambda i: (0, 0)),
    ],
    out_shape=[
        jax.ShapeDtypeStruct((S, 1), jnp.int32),
        jax.ShapeDtypeStruct((1, 1), jnp.float32),
        jax.ShapeDtypeStruct((1, E), jnp.int32),
        jax.ShapeDtypeStruct((1, E), jnp.int32),
    ],
    scratch_shapes=[
        pltpu.VMEM((1, E), jnp.float32),
        pltpu.VMEM((1, E), jnp.float32),
        pltpu.VMEM((1, 1), jnp.float32),
    ],
)


def _make_dispatch_body(half):
    def body(x_hbm, pk_hbm, qc_hbm, xd_hbm, gps_hbm, tfso_hbm,
             apk_v, tab_v, dtok_v, qc_v, idxa_v, idxb_v, gps_v, tfso_v,
             ra_v, rb_v, semg, semwa, semwb):
        wid = lax.axis_index("s") * NC + lax.axis_index("c")
        lbase = wid * SPW
        base = half * HC + lbase
        e = base >> 7

        pltpu.sync_copy(pk_hbm, apk_v)
        pltpu.sync_copy(qc_hbm, qc_v)

        def init_body(j, _):
            tab_v[pl.ds(j * 16, 16)] = jnp.zeros((16,), jnp.int32)
            return 0

        lax.fori_loop(0, SPW // 16, init_body, 0)

        def scat_body(j, _):
            pk = apk_v[pl.ds(j * 16, 16)]
            sl = pk & 4095
            d = sl - base
            tok = lax.iota(jnp.int32, 16) + j * 16
            m = (d >= 0) & (d < SPW)
            plsc.store_scatter(tab_v, [d & (SPW - 1)], (pk & _GMASK) | tok,
                               mask=m)
            dr = (pk >> 12) & (S - 1)
            plsc.store_scatter(dtok_v, [dr], tok, mask=sl == 4095)
            return 0

        lax.fori_loop(0, S // 16, scat_body, 0)

        esplat = lax.iota(jnp.int32, 16) * 0 + e
        qcw = plsc.load_gather(qc_v, [esplat])
        cnt_e = qcw & 255
        q_e = qcw >> 8
        for j, idx_v in enumerate((idxa_v, idxb_v)):
            w = tab_v[pl.ds(j * 16, 16)]
            tok_tab = w & 4095
            idx_v[...] = tok_tab
            gps_v[pl.ds(j * 16, 16)] = lax.bitcast_convert_type(
                w & _GMASK, jnp.float32)
            t = lax.iota(jnp.int32, 16) + (base + j * 16)
            filled = (t & 127) < cnt_e
            re = (t - q_e) & (S - 1)
            dt = plsc.load_gather(dtok_v, [re])
            tfso_v[pl.ds(j * 16, 16)] = jnp.where(filled, tok_tab, dt)
        pltpu.sync_copy(gps_v, gps_hbm.at[pl.ds(lbase, SPW)])
        pltpu.sync_copy(tfso_v, tfso_hbm.at[pl.ds(lbase, SPW)])
        pltpu.async_copy(x_hbm.at[idxa_v], ra_v, semg).wait()
        wa = pltpu.async_copy(ra_v, xd_hbm.at[pl.ds(lbase, 16)], semwa)
        pltpu.async_copy(x_hbm.at[idxb_v], rb_v, semg).wait()
        wb = pltpu.async_copy(rb_v, xd_hbm.at[pl.ds(lbase + 16, 16)], semwb)
        wa.wait()
        wb.wait()

    return body


def _combine_body(ys_hbm, tfso_hbm, out_hbm,
                  idxa_v, idxb_v, ra_v, rb_v, semg, semwa, semwb):
    wid = lax.axis_index("s") * NC + lax.axis_index("c")
    lbase = wid * SPW

    pltpu.sync_copy(tfso_hbm.at[pl.ds(lbase, 16)], idxa_v)
    pltpu.sync_copy(tfso_hbm.at[pl.ds(lbase + 16, 16)], idxb_v)
    pltpu.async_copy(ys_hbm.at[pl.ds(lbase, 16)], ra_v, semg).wait()
    wa = pltpu.async_copy(ra_v, out_hbm.at[idxa_v], semwa)
    pltpu.async_copy(ys_hbm.at[pl.ds(lbase + 16, 16)], rb_v, semg).wait()
    wb = pltpu.async_copy(rb_v, out_hbm.at[idxb_v], semwb)
    wa.wait()
    wb.wait()


@functools.cache
def _sc_kernels():
    mesh = plsc.VectorSubcoreMesh(core_axis_name="c", subcore_axis_name="s",
                                  num_cores=NC, num_subcores=NS)
    params = pltpu.CompilerParams(needs_layout_passes=False)
    dispatches = [
        pl.kernel(
            _make_dispatch_body(h),
            out_type=[
                jax.ShapeDtypeStruct((HC, D), jnp.float32),
                jax.ShapeDtypeStruct((HC,), jnp.float32),
                jax.ShapeDtypeStruct((HC,), jnp.int32),
            ],
            mesh=mesh,
            compiler_params=params,
            name=f"dispatch{h}",
            scratch_types=[
                pltpu.VMEM((S,), jnp.int32),
                pltpu.VMEM((SPW,), jnp.int32),
                pltpu.VMEM((S,), jnp.int32),
                pltpu.VMEM((16,), jnp.int32),
                pltpu.VMEM((16,), jnp.int32),
                pltpu.VMEM((16,), jnp.int32),
                pltpu.VMEM((SPW,), jnp.float32),
                pltpu.VMEM((SPW,), jnp.int32),
                pltpu.VMEM((16, D), jnp.float32),
                pltpu.VMEM((16, D), jnp.float32),
                pltpu.SemaphoreType.DMA,
                pltpu.SemaphoreType.DMA,
                pltpu.SemaphoreType.DMA,
            ],
        )
        for h in (0, 1)
    ]
    combine = pl.kernel(
        _combine_body,
        mesh=mesh,
        compiler_params=params,
        name="combine",
        scratch_types=[
            pltpu.VMEM((16,), jnp.int32),
            pltpu.VMEM((16,), jnp.int32),
            pltpu.VMEM((16, D), jnp.float32),
            pltpu.VMEM((16, D), jnp.float32),
            pltpu.SemaphoreType.DMA,
            pltpu.SemaphoreType.DMA,
            pltpu.SemaphoreType.DMA,
        ],
    )
    return dispatches, combine


def _mlp_body(xd_ref, w1_ref, b1_ref, w2_ref, b2_ref, gps_ref, out_ref):
    xb = xd_ref[0]
    h = jnp.dot(xb, w1_ref[0], preferred_element_type=jnp.float32) + b1_ref[0]
    h = jax.nn.gelu(h)
    y = jnp.dot(h, w2_ref[0], preferred_element_type=jnp.float32) + b2_ref[0]
    out_ref[0] = y * gps_ref[0]


def _make_mlp(half):
    off = half * EH
    return pl.pallas_call(
        _mlp_body,
        grid=(EH,),
        in_specs=[
            pl.BlockSpec((1, C, D), lambda e: (e, 0, 0)),
            pl.BlockSpec((1, D, F), lambda e, o=off: (e + o, 0, 0)),
            pl.BlockSpec((1, 1, F), lambda e, o=off: (e + o, 0, 0)),
            pl.BlockSpec((1, F, D), lambda e, o=off: (e + o, 0, 0)),
            pl.BlockSpec((1, 1, D), lambda e, o=off: (e + o, 0, 0)),
            pl.BlockSpec((1, C, 1), lambda e: (e, 0, 0)),
        ],
        out_specs=pl.BlockSpec((1, C, D), lambda e: (e, 0, 0)),
        out_shape=jax.ShapeDtypeStruct((EH, C, D), jnp.float32),
    )


_mlp_a = _make_mlp(0)
_mlp_b = _make_mlp(1)


def kernel(hidden_states, wg, w1, b1, w2, b2):
    x = hidden_states.reshape(S, D)
    pk2, laux, cnt2, qc2 = _router(x, wg)
    pk = pk2.reshape(S)
    cnt = cnt2.reshape(E)
    qc = qc2.reshape(E)
    (_dispatch_a, _dispatch_b), _combine = _sc_kernels()
    xda, gpsa, tfsoa = _dispatch_a(x, pk, qc)
    pk_b, xda = lax.optimization_barrier((pk, xda))
    xdb, gpsb, tfsob = _dispatch_b(x, pk_b, qc)
    b1r = b1.reshape(E, 1, F)
    b2r = b2.reshape(E, 1, D)
    ysa = _mlp_a(xda.reshape(EH, C, D), w1, b1r, w2, b2r,
                 gpsa.reshape(EH, C, 1))
    ysb = _mlp_b(xdb.reshape(EH, C, D), w1, b1r, w2, b2r,
                 gpsb.reshape(EH, C, 1))
    out_ref = jax.new_ref(jnp.zeros((S, D), jnp.float32))
    _combine(ysa.reshape(HC, D), tfsoa, out_ref)
    _combine(ysb.reshape(HC, D), tfsob, out_ref)
    out = out_ref[...]
    return out.reshape(hidden_states.shape), laux.reshape(()), cnt


# speedup vs baseline: 1.0215x; 1.0215x over previous
import functools

import jax
import jax.numpy as jnp
from jax import lax
from jax.experimental import pallas as pl
from jax.experimental.pallas import tpu as pltpu
from jax.experimental.pallas import tpu_sc as plsc

S = 2048
D = 1024
E = 16
EH = E // 2
F = 1024
C = 128
EC = E * C
HC = EC // 2
RB = 512
NR = S // RB
NC = 2
NS = 16
NW = NC * NS
SPW = HC // NW

_GMASK = -4096


def _router_body(x_ref, wg_ref, pk_ref, laux_ref, cnt_ref, qc_ref,
                 carry_ref, acc_ref, dcarry_ref):
    i = pl.program_id(0)

    @pl.when(i == 0)
    def _():
        carry_ref[...] = jnp.zeros((1, E), jnp.float32)
        acc_ref[...] = jnp.zeros((1, E), jnp.float32)
        dcarry_ref[...] = jnp.zeros((1, 1), jnp.float32)

    x = x_ref[...]
    wg = wg_ref[...]
    logits = jnp.dot(x, wg, preferred_element_type=jnp.float32)
    mx = jnp.max(logits, axis=1, keepdims=True)
    p = jnp.exp(logits - mx)
    gates = p / jnp.sum(p, axis=1, keepdims=True)
    gmax = jnp.max(gates, axis=1, keepdims=True)
    ie = lax.broadcasted_iota(jnp.int32, (RB, E), 1)
    idx1 = jnp.min(jnp.where(gates == gmax, ie, E), axis=1, keepdims=True)
    oh = (ie == idx1).astype(jnp.float32)

    carry = carry_ref[...]
    tri = (lax.broadcasted_iota(jnp.int32, (RB, RB), 0) >=
           lax.broadcasted_iota(jnp.int32, (RB, RB), 1)).astype(jnp.float32)
    incl = jnp.dot(tri, oh, preferred_element_type=jnp.float32) + carry
    pos = incl - 1.0
    pos_s = jnp.sum(pos * oh, axis=1, keepdims=True)
    kept = pos_s < C

    dmask = jnp.where(kept, 0.0, 1.0)
    dincl = (jnp.dot(tri, dmask, preferred_element_type=jnp.float32)
             + dcarry_ref[...])
    drank = (dincl - 1.0).astype(jnp.int32)
    dcarry_ref[...] = dcarry_ref[...] + jnp.sum(dmask, axis=0, keepdims=True)

    gbits = lax.bitcast_convert_type(gmax, jnp.int32) & _GMASK
    slot = idx1 * C + pos_s.astype(jnp.int32)
    pkw = jnp.where(kept, gbits | slot, (drank << 12) | 4095)
    pk_ref[...] = pkw.reshape(1, RB // 128, 128)

    counts = carry + jnp.sum(oh, axis=0, keepdims=True)
    carry_ref[...] = counts
    me = acc_ref[...] + jnp.sum(gates, axis=0, keepdims=True)
    acc_ref[...] = me

    @pl.when(i == NR - 1)
    def _():
        cnt_post = jnp.minimum(counts, C)
        cnt_ref[...] = cnt_post.astype(jnp.int32)
        laux_ref[...] = jnp.sum(me * counts, axis=1,
                                keepdims=True) * (E / (S * S))
        ut = (lax.broadcasted_iota(jnp.int32, (E, E), 0) <=
              lax.broadcasted_iota(jnp.int32, (E, E), 1)).astype(jnp.float32)
        q = jnp.dot(cnt_post, ut,
                    preferred_element_type=jnp.float32).astype(jnp.int32)
        qc_ref[...] = (q << 8) | cnt_post.astype(jnp.int32)


_router = pl.pallas_call(
    _router_body,
    grid=(NR,),
    in_specs=[
        pl.BlockSpec((RB, D), lambda i: (i, 0)),
        pl.BlockSpec((D, E), lambda i: (0, 0)),
    ],
    out_specs=[
        pl.BlockSpec((1, RB // 128, 128), lambda i: (i, 0, 0)),
        pl.BlockSpec((1, 1), lambda i: (0, 0)),
        pl.BlockSpec((1, E), lambda i: (0, 0)),
        pl.BlockSpec((1, E), lambda i: (0, 0)),
    ],
    out_shape=[
        jax.ShapeDtypeStruct((NR, RB // 128, 128), jnp.int32),
        jax.ShapeDtypeStruct((1, 1), jnp.float32),
        jax.ShapeDtypeStruct((1, E), jnp.int32),
        jax.ShapeDtypeStruct((1, E), jnp.int32),
    ],
    scratch_shapes=[
        pltpu.VMEM((1, E), jnp.float32),
        pltpu.VMEM((1, E), jnp.float32),
        pltpu.VMEM((1, 1), jnp.float32),
    ],
)


def _make_dispatch_body(half):
    def body(x_hbm, pk_hbm, qc_hbm, xd_hbm, gps_hbm, tfso_hbm,
             apk_v, tab_v, dtok_v, qc_v, idxa_v, idxb_v, gps_v, tfso_v,
             ra_v, rb_v, semg, semwa, semwb):
        wid = lax.axis_index("s") * NC + lax.axis_index("c")
        lbase = wid * SPW
        base = half * HC + lbase
        e = base >> 7

        pltpu.sync_copy(pk_hbm, apk_v)
        pltpu.sync_copy(qc_hbm, qc_v)

        def init_body(j, _):
            tab_v[pl.ds(j * 16, 16)] = jnp.zeros((16,), jnp.int32)
            return 0

        lax.fori_loop(0, SPW // 16, init_body, 0)

        def scat_body(j, _):
            pk = apk_v[pl.ds(j * 16, 16)]
            sl = pk & 4095
            d = sl - base
            tok = lax.iota(jnp.int32, 16) + j * 16
            m = (d >= 0) & (d < SPW)
            plsc.store_scatter(tab_v, [d & (SPW - 1)], (pk & _GMASK) | tok,
                               mask=m)
            dr = (pk >> 12) & (S - 1)
            plsc.store_scatter(dtok_v, [dr], tok, mask=sl == 4095)
            return 0

        lax.fori_loop(0, S // 16, scat_body, 0)

        esplat = lax.iota(jnp.int32, 16) * 0 + e
        qcw = plsc.load_gather(qc_v, [esplat])
        cnt_e = qcw & 255
        q_e = qcw >> 8
        for j, idx_v in enumerate((idxa_v, idxb_v)):
            w = tab_v[pl.ds(j * 16, 16)]
            tok_tab = w & 4095
            idx_v[...] = tok_tab
            gps_v[pl.ds(j * 16, 16)] = lax.bitcast_convert_type(
                w & _GMASK, jnp.float32)
            t = lax.iota(jnp.int32, 16) + (base + j * 16)
            filled = (t & 127) < cnt_e
            re = (t - q_e) & (S - 1)
            dt = plsc.load_gather(dtok_v, [re])
            tfso_v[pl.ds(j * 16, 16)] = jnp.where(filled, tok_tab, dt)
        pltpu.sync_copy(gps_v, gps_hbm.at[pl.ds(lbase, SPW)])
        pltpu.sync_copy(tfso_v, tfso_hbm.at[pl.ds(lbase, SPW)])
        pltpu.async_copy(x_hbm.at[idxa_v], ra_v, semg).wait()
        wa = pltpu.async_copy(ra_v, xd_hbm.at[pl.ds(lbase, 16)], semwa)
        pltpu.async_copy(x_hbm.at[idxb_v], rb_v, semg).wait()
        wb = pltpu.async_copy(rb_v, xd_hbm.at[pl.ds(lbase + 16, 16)], semwb)
        wa.wait()
        wb.wait()

    return body


def _combine_body(ys_hbm, tfso_hbm, out_hbm,
                  idxa_v, idxb_v, ra_v, rb_v, semg, semwa, semwb):
    wid = lax.axis_index("s") * NC + lax.axis_index("c")
    lbase = wid * SPW

    pltpu.sync_copy(tfso_hbm.at[pl.ds(lbase, 16)], idxa_v)
    pltpu.sync_copy(tfso_hbm.at[pl.ds(lbase + 16, 16)], idxb_v)
    pltpu.async_copy(ys_hbm.at[pl.ds(lbase, 16)], ra_v, semg).wait()
    wa = pltpu.async_copy(ra_v, out_hbm.at[idxa_v], semwa)
    pltpu.async_copy(ys_hbm.at[pl.ds(lbase + 16, 16)], rb_v, semg).wait()
    wb = pltpu.async_copy(rb_v, out_hbm.at[idxb_v], semwb)
    wa.wait()
    wb.wait()


@functools.cache
def _sc_kernels():
    mesh = plsc.VectorSubcoreMesh(core_axis_name="c", subcore_axis_name="s",
                                  num_cores=NC, num_subcores=NS)
    params = pltpu.CompilerParams(needs_layout_passes=False)
    dispatches = [
        pl.kernel(
            _make_dispatch_body(h),
            out_type=[
                jax.ShapeDtypeStruct((HC, D), jnp.float32),
                jax.ShapeDtypeStruct((HC,), jnp.float32),
                jax.ShapeDtypeStruct((HC,), jnp.int32),
            ],
            mesh=mesh,
            compiler_params=params,
            name=f"dispatch{h}",
            scratch_types=[
                pltpu.VMEM((S,), jnp.int32),
                pltpu.VMEM((SPW,), jnp.int32),
                pltpu.VMEM((S,), jnp.int32),
                pltpu.VMEM((16,), jnp.int32),
                pltpu.VMEM((16,), jnp.int32),
                pltpu.VMEM((16,), jnp.int32),
                pltpu.VMEM((SPW,), jnp.float32),
                pltpu.VMEM((SPW,), jnp.int32),
                pltpu.VMEM((16, D), jnp.float32),
                pltpu.VMEM((16, D), jnp.float32),
                pltpu.SemaphoreType.DMA,
                pltpu.SemaphoreType.DMA,
                pltpu.SemaphoreType.DMA,
            ],
        )
        for h in (0, 1)
    ]
    combine = pl.kernel(
        _combine_body,
        mesh=mesh,
        compiler_params=params,
        name="combine",
        scratch_types=[
            pltpu.VMEM((16,), jnp.int32),
            pltpu.VMEM((16,), jnp.int32),
            pltpu.VMEM((16, D), jnp.float32),
            pltpu.VMEM((16, D), jnp.float32),
            pltpu.SemaphoreType.DMA,
            pltpu.SemaphoreType.DMA,
            pltpu.SemaphoreType.DMA,
        ],
    )
    return dispatches, combine


def _mlp_body(xd_ref, w1_ref, b1_ref, w2_ref, b2_ref, gps_ref, out_ref):
    xb = xd_ref[0]
    h = jnp.dot(xb, w1_ref[0], preferred_element_type=jnp.float32) + b1_ref[0]
    h = jax.nn.gelu(h)
    y = jnp.dot(h, w2_ref[0], preferred_element_type=jnp.float32) + b2_ref[0]
    out_ref[0] = y * gps_ref[0]


def _make_mlp(half):
    off = half * EH
    return pl.pallas_call(
        _mlp_body,
        grid=(EH,),
        in_specs=[
            pl.BlockSpec((1, C, D), lambda e: (e, 0, 0)),
            pl.BlockSpec((1, D, F), lambda e, o=off: (e + o, 0, 0)),
            pl.BlockSpec((1, 1, F), lambda e, o=off: (e + o, 0, 0)),
            pl.BlockSpec((1, F, D), lambda e, o=off: (e + o, 0, 0)),
            pl.BlockSpec((1, 1, D), lambda e, o=off: (e + o, 0, 0)),
            pl.BlockSpec((1, C, 1), lambda e: (e, 0, 0)),
        ],
        out_specs=pl.BlockSpec((1, C, D), lambda e: (e, 0, 0)),
        out_shape=jax.ShapeDtypeStruct((EH, C, D), jnp.float32),
    )


_mlp_a = _make_mlp(0)
_mlp_b = _make_mlp(1)


def kernel(hidden_states, wg, w1, b1, w2, b2):
    x = hidden_states.reshape(S, D)
    pk2, laux, cnt2, qc2 = _router(x, wg)
    pk = pk2.reshape(S)
    cnt = cnt2.reshape(E)
    qc = qc2.reshape(E)
    (_dispatch_a, _dispatch_b), _combine = _sc_kernels()
    xda, gpsa, tfsoa = _dispatch_a(x, pk, qc)
    outz = jnp.zeros((S, D), jnp.float32)
    pk_b, xda, outz = lax.optimization_barrier((pk, xda, outz))
    xdb, gpsb, tfsob = _dispatch_b(x, pk_b, qc)
    b1r = b1.reshape(E, 1, F)
    b2r = b2.reshape(E, 1, D)
    ysa = _mlp_a(xda.reshape(EH, C, D), w1, b1r, w2, b2r,
                 gpsa.reshape(EH, C, 1))
    ysb = _mlp_b(xdb.reshape(EH, C, D), w1, b1r, w2, b2r,
                 gpsb.reshape(EH, C, 1))
    out_ref = jax.new_ref(outz)
    _combine(ysa.reshape(HC, D), tfsoa, out_ref)
    _combine(ysb.reshape(HC, D), tfsob, out_ref)
    out = out_ref[...]
    return out.reshape(hidden_states.shape), laux.reshape(()), cnt
